# baseline (speedup 1.0000x reference)
import functools

import jax
import jax.numpy as jnp
from jax import lax
from jax.experimental import pallas as pl
from jax.experimental.pallas import tpu as pltpu
from jax.experimental.pallas import tpu_sc as plsc

_B, _S, _D, _V = 4, 4096, 1024, 100000
_L = 16
_NC, _NS = 2, 16
_NW = _NC * _NS
_TOK = _B * _S
_TPW = _TOK // _NW
_C = 16
_NCH = _TPW // _C
_VREGS = _D // _L
_SCALE = float(_D) ** 0.5


def _pe_table():
    pos = jnp.arange(_S, dtype=jnp.float32)[:, None]
    i = jnp.arange(0, _D, 2, dtype=jnp.float32)
    angle = pos / jnp.power(10000.0, i / _D)
    pe = jnp.zeros((_S, _D), dtype=jnp.float32)
    pe = pe.at[:, 0::2].set(jnp.sin(angle))
    pe = pe.at[:, 1::2].set(jnp.cos(angle))
    return pe


def _sc_embed(x_flat, table, pe, gamma, beta):
    mesh = plsc.VectorSubcoreMesh(core_axis_name="c", subcore_axis_name="s")

    @functools.partial(
        pl.kernel,
        mesh=mesh,
        out_type=jax.ShapeDtypeStruct((_TOK, _D), jnp.float32),
        scratch_types=[
            pltpu.VMEM((_C,), jnp.int32),
            pltpu.VMEM((_C, _D), jnp.float32),
            pltpu.VMEM((_C, _D), jnp.float32),
            pltpu.VMEM((_C, _L), jnp.float32),
            pltpu.VMEM((_C, _L), jnp.float32),
            pltpu.VMEM((_D,), jnp.float32),
            pltpu.VMEM((_D,), jnp.float32),
            pltpu.SemaphoreType.DMA,
        ],
        compiler_params=pltpu.CompilerParams(needs_layout_passes=False),
    )
    def k(x_hbm, table_hbm, pe_hbm, gamma_hbm, beta_hbm, out_hbm,
          idx_v, rows_v, pe_v, sums_v, sums2_v, g_v, b_v, sem):
        wid = lax.axis_index("s") * _NC + lax.axis_index("c")
        tok0 = wid * _TPW
        pos0 = tok0 % _S
        pltpu.sync_copy(gamma_hbm, g_v)
        pltpu.sync_copy(beta_hbm, b_v)
        riota = jax.lax.broadcasted_iota(jnp.int32, (_L,), 0)

        def chunk_body(c, _):
            base = tok0 + c * _C
            pos = pos0 + c * _C
            pltpu.sync_copy(x_hbm.at[pl.ds(base, _C)], idx_v)
            pltpu.async_copy(table_hbm.at[idx_v], rows_v, sem).wait()
            pltpu.sync_copy(pe_hbm.at[pl.ds(pos, _C)], pe_v)

            def row_body(r, _):
                def p1(j, carry):
                    acc, acc2 = carry
                    sl = pl.ds(j * _L, _L)
                    v = rows_v[r, sl] * _SCALE + pe_v[r, sl]
                    rows_v[r, sl] = v
                    return acc + v, acc2 + v * v

                z = jnp.zeros((_L,), jnp.float32)
                acc, acc2 = lax.fori_loop(0, _VREGS, p1, (z, z), unroll=4)
                sums_v[r, :] = acc
                sums2_v[r, :] = acc2
                return 0

            lax.fori_loop(0, _C, row_body, 0)

            s1 = jnp.zeros((_L,), jnp.float32)
            s2 = jnp.zeros((_L,), jnp.float32)
            for cc in range(_L):
                col = jnp.full((_L,), cc, jnp.int32)
                s1 = s1 + plsc.load_gather(sums_v, [riota, col])
                s2 = s2 + plsc.load_gather(sums2_v, [riota, col])
            mean = s1 * (1.0 / _D)
            var = s2 * (1.0 / _D) - mean * mean
            xv = var + 1e-5
            iv = lax.bitcast_convert_type(xv, jnp.int32)
            iv = jnp.int32(0x5F3759DF) - lax.shift_right_logical(iv, 1)
            y = lax.bitcast_convert_type(iv, jnp.float32)
            y = y * (1.5 - 0.5 * xv * y * y)
            y = y * (1.5 - 0.5 * xv * y * y)
            y = y * (1.5 - 0.5 * xv * y * y)

            def p2(j, _):
                gvec = g_v[pl.ds(j * _L, _L)]
                bvec = b_v[pl.ds(j * _L, _L)]
                d0 = j * _L
                for t in range(_L):
                    col = jnp.full((_L,), d0 + t, jnp.int32)
                    v = plsc.load_gather(rows_v, [riota, col])
                    o = (v - mean) * y * gvec[t] + bvec[t]
                    plsc.store_scatter(rows_v, [riota, col], o)
                return 0

            lax.fori_loop(0, _VREGS, p2, 0)
            pltpu.sync_copy(rows_v, out_hbm.at[pl.ds(base, _C)])
            return 0

        lax.fori_loop(0, _NCH, chunk_body, 0)

    return k(x_flat, table, pe, gamma, beta)


def kernel(x, table, gamma, beta):
    pe = _pe_table()
    out = _sc_embed(x.reshape(-1), table, pe, gamma, beta)
    return out.reshape(_B, _S, _D)
